# Initial kernel scaffold; baseline (speedup 1.0000x reference)
import jax
import jax.numpy as jnp
from jax.experimental import pallas as pl


def kernel(z, emb_weight):
    raise NotImplementedError("write your pallas kernel here")


# speedup vs baseline: 1.0107x; 1.0107x over previous
import jax
import jax.numpy as jnp
from jax.experimental import pallas as pl

_BETA = 0.25
_BT = 1024


def _vq_block(z_ref, zsq_ref, e_ref, zq_ref, loss_ref):
    zb = z_ref[...]
    e = e_ref[...]
    zsq = zsq_ref[...]
    esq = jnp.sum(e * e, axis=1)
    mm = jax.lax.dot_general(zb, e, (((1,), (1,)), ((), ())),
                             preferred_element_type=jnp.float32)
    d = (zsq + esq[None, :]) - 2.0 * mm
    dmin = jnp.min(d, axis=1, keepdims=True)
    iota = jax.lax.broadcasted_iota(jnp.int32, d.shape, 1)
    big = jnp.int32(d.shape[1])
    idx = jnp.min(jnp.where(d == dmin, iota, big), axis=1)
    onehot = (iota == idx[:, None]).astype(jnp.float32)
    zq = jax.lax.dot_general(onehot, e, (((1,), (0,)), ((), ())),
                             preferred_element_type=jnp.float32,
                             precision=jax.lax.Precision.HIGHEST)
    zq_ref[...] = zq
    diff = zq - zb
    loss_ref[...] = jnp.full((1, 1, 128), jnp.sum(diff * diff), jnp.float32)


def kernel(z, emb_weight):
    B, C, H, W = z.shape
    N, D = emb_weight.shape
    zp = jnp.transpose(z, (0, 2, 3, 1))
    z_flat = zp.reshape(-1, D)
    zsq = jnp.sum(z_flat ** 2, axis=1, keepdims=True)
    T = z_flat.shape[0]
    nblk = T // _BT

    zq_flat, loss_parts = pl.pallas_call(
        _vq_block,
        grid=(nblk,),
        in_specs=[
            pl.BlockSpec((_BT, D), lambda i: (i, 0)),
            pl.BlockSpec((_BT, 1), lambda i: (i, 0)),
            pl.BlockSpec((N, D), lambda i: (0, 0)),
        ],
        out_specs=[
            pl.BlockSpec((_BT, D), lambda i: (i, 0)),
            pl.BlockSpec((1, 1, 128), lambda i: (i, 0, 0)),
        ],
        out_shape=[
            jax.ShapeDtypeStruct((T, D), jnp.float32),
            jax.ShapeDtypeStruct((nblk, 1, 128), jnp.float32),
        ],
    )(z_flat, zsq, emb_weight)

    sq_sum = jnp.sum(loss_parts[:, 0, 0])
    mean_sq = sq_sum / (T * D)
    loss = _BETA * mean_sq + mean_sq
    z_quantise = jnp.transpose(zq_flat.reshape(zp.shape), (0, 3, 1, 2))
    return (z_quantise, loss)


# speedup vs baseline: 1.0336x; 1.0227x over previous
import jax
import jax.numpy as jnp
from jax.experimental import pallas as pl

_BETA = 0.25


def _vq_block(z_ref, e_ref, zq_ref, loss_ref):
    zct = z_ref[0]
    e = e_ref[...]
    n = e.shape[0]
    zsq = jnp.sum(zct * zct, axis=0, keepdims=True)
    esq = jnp.sum(e * e, axis=1)
    mm = jax.lax.dot_general(e, zct, (((1,), (0,)), ((), ())),
                             preferred_element_type=jnp.float32)
    dist = (zsq + esq[:, None]) - 2.0 * mm
    dmin = jnp.min(dist, axis=0, keepdims=True)
    iota = jax.lax.broadcasted_iota(jnp.int32, dist.shape, 0)
    idx = jnp.min(jnp.where(dist == dmin, iota, jnp.int32(n)),
                  axis=0)
    onehot = (iota == idx[None, :]).astype(jnp.float32)
    zq = jax.lax.dot_general(e, onehot, (((0,), (0,)), ((), ())),
                             preferred_element_type=jnp.float32,
                             precision=jax.lax.Precision.HIGHEST)
    zq_ref[...] = zq[None]
    diff = zq - zct
    loss_ref[...] = jnp.full((1, 1, 128), jnp.sum(diff * diff), jnp.float32)


def kernel(z, emb_weight):
    B, C, H, W = z.shape
    N, D = emb_weight.shape
    T = H * W
    zr = z.reshape(B, C, T)

    zq_r, loss_parts = pl.pallas_call(
        _vq_block,
        grid=(B,),
        in_specs=[
            pl.BlockSpec((1, C, T), lambda i: (i, 0, 0)),
            pl.BlockSpec((N, D), lambda i: (0, 0)),
        ],
        out_specs=[
            pl.BlockSpec((1, C, T), lambda i: (i, 0, 0)),
            pl.BlockSpec((1, 1, 128), lambda i: (i, 0, 0)),
        ],
        out_shape=[
            jax.ShapeDtypeStruct((B, C, T), jnp.float32),
            jax.ShapeDtypeStruct((B, 1, 128), jnp.float32),
        ],
    )(zr, emb_weight)

    sq_sum = jnp.sum(loss_parts[:, 0, 0])
    mean_sq = sq_sum / (B * T * D)
    loss = _BETA * mean_sq + mean_sq
    return (zq_r.reshape(B, C, H, W), loss)
